# Initial kernel scaffold; baseline (speedup 1.0000x reference)
import jax
import jax.numpy as jnp
from jax.experimental import pallas as pl


def kernel(adj_norm, adj_ori, feats_ori, img_feats, csd_ori, csd_img, Wb, Wm, Wl, W1, b1, W2, b2):
    raise NotImplementedError("write your pallas kernel here")


# speedup vs baseline: 25.6390x; 25.6390x over previous
import functools

import jax
import jax.numpy as jnp
from jax import lax
from jax.experimental import pallas as pl
from jax.experimental.pallas import tpu as pltpu
from jax.experimental.pallas import tpu_sc as plsc

N = 4096
BLK = 256
GR = N // BLK

NW = 32
CHUNK = 8192
NCHUNK = N * N // NW // CHUNK
NBIN = 1024
HIST = 2 * NBIN * 16


def _mm_body(a_ref, b_ref, o_ref):
    o_ref[:] = jnp.dot(a_ref[:], b_ref[:], preferred_element_type=jnp.float32)


def _mm(a, b):
    return pl.pallas_call(
        _mm_body,
        out_shape=jax.ShapeDtypeStruct((a.shape[0], b.shape[1]), jnp.float32),
    )(a, b)


def _bigmm_body(a_ref, b_ref, o_ref, *, relu):
    r = jnp.dot(a_ref[:], b_ref[:], preferred_element_type=jnp.float32)
    if relu:
        r = jnp.maximum(r, 0.0)
    o_ref[:] = r


def _bigmm(a, b, relu=False):
    k = b.shape[1]
    return pl.pallas_call(
        functools.partial(_bigmm_body, relu=relu),
        grid=(GR,),
        in_specs=[
            pl.BlockSpec((BLK, N), lambda i: (i, 0)),
            pl.BlockSpec((N, k), lambda i: (0, 0)),
        ],
        out_specs=pl.BlockSpec((BLK, k), lambda i: (i, 0)),
        out_shape=jax.ShapeDtypeStruct((N, k), jnp.float32),
    )(a, b)


def _logits_body(mi_ref, mj_ref, o_ref, mx_ref):
    gi = pl.program_id(0)
    gj = pl.program_id(1)
    lg = lax.dot_general(mi_ref[:], mj_ref[:], (((1,), (1,)), ((), ())),
                         preferred_element_type=jnp.float32)
    o_ref[:] = lg
    rows = gi * BLK + lax.broadcasted_iota(jnp.int32, (BLK, BLK), 0)
    cols = gj * BLK + lax.broadcasted_iota(jnp.int32, (BLK, BLK), 1)
    bm = jnp.max(jnp.where(cols > rows, lg, -jnp.inf))

    @pl.when(jnp.logical_and(gi == 0, gj == 0))
    def _():
        mx_ref[:] = jnp.full_like(mx_ref, -jnp.inf)

    mx_ref[:] = jnp.maximum(mx_ref[:], bm)


def _logits(mean):
    return pl.pallas_call(
        _logits_body,
        grid=(GR, GR),
        in_specs=[
            pl.BlockSpec((BLK, 64), lambda i, j: (i, 0)),
            pl.BlockSpec((BLK, 64), lambda i, j: (j, 0)),
        ],
        out_specs=[
            pl.BlockSpec((BLK, BLK), lambda i, j: (i, j)),
            pl.BlockSpec((8, 128), lambda i, j: (0, 0)),
        ],
        out_shape=[
            jax.ShapeDtypeStruct((N, N), jnp.float32),
            jax.ShapeDtypeStruct((8, 128), jnp.float32),
        ],
    )(mean, mean)


def _key_body(lg_ref, adj_ref, mx_ref, key_ref, cnt_ref):
    gi = pl.program_id(0)
    maxv = mx_ref[0]
    lg = lg_ref[:]
    adj = adj_ref[:]
    ep = lg / maxv
    rows = gi * BLK + lax.broadcasted_iota(jnp.int32, (BLK, N), 0)
    cols = lax.broadcasted_iota(jnp.int32, (BLK, N), 1)
    valid = jnp.logical_and(cols > rows, ep > 0.0)
    bits = lax.bitcast_convert_type(ep, jnp.int32)
    keyed = jnp.where(adj == 0.0, bits | jnp.int32(-2147483648), bits)
    key_ref[:] = jnp.where(valid, keyed, 0)
    cnt = jnp.sum((adj != 0.0).astype(jnp.int32))

    @pl.when(gi == 0)
    def _():
        cnt_ref[:] = jnp.zeros_like(cnt_ref)

    cnt_ref[:] = cnt_ref[:] + cnt


def _keys(logits, adj_ori, maxv):
    return pl.pallas_call(
        _key_body,
        grid=(GR,),
        in_specs=[
            pl.BlockSpec((BLK, N), lambda i: (i, 0)),
            pl.BlockSpec((BLK, N), lambda i: (i, 0)),
            pl.BlockSpec(memory_space=pltpu.SMEM),
        ],
        out_specs=[
            pl.BlockSpec((BLK, N), lambda i: (i, 0)),
            pl.BlockSpec((8, 128), lambda i: (0, 0)),
        ],
        out_shape=[
            jax.ShapeDtypeStruct((N, N), jnp.int32),
            jax.ShapeDtypeStruct((8, 128), jnp.int32),
        ],
    )(logits, adj_ori, maxv)


def _sc_hist_body(key_hbm, par_hbm, hist_hbm, keybuf, parbuf, histbuf):
    c = lax.axis_index("c")
    s = lax.axis_index("s")
    wid = s * 2 + c
    pltpu.sync_copy(par_hbm, parbuf)

    def zero_body(i, carry):
        histbuf[pl.ds(i * 16, 16)] = jnp.zeros((16,), jnp.int32)
        return carry

    lax.fori_loop(0, HIST // 16, zero_body, 0)

    p_rm = parbuf[pl.ds(0, 16)]
    p_add = parbuf[pl.ds(16, 16)]
    s_p = parbuf[pl.ds(32, 16)]
    s_d = parbuf[pl.ds(48, 16)]
    lane = lax.iota(jnp.int32, 16)
    ones = jnp.ones((16,), jnp.int32)

    def chunk_body(ci, carry):
        pltpu.sync_copy(key_hbm.at[wid, ci], keybuf)

        def vec_body(j, carry2):
            k = keybuf[pl.ds(j * 16, 16)]
            bits = lax.bitwise_and(k, jnp.int32(0x3FFFFFFF))
            pref = lax.shift_right_logical(bits, s_p)
            dig = lax.bitwise_and(lax.shift_right_logical(bits, s_d),
                                  jnp.int32(NBIN - 1))
            m_rm = jnp.logical_and(k > 0, pref == p_rm)
            m_add = jnp.logical_and(k < 0, pref == p_add)
            idx_rm = dig * 16 + lane
            idx_add = idx_rm + NBIN * 16
            plsc.addupdate_scatter(histbuf, [idx_rm], ones, mask=m_rm)
            plsc.addupdate_scatter(histbuf, [idx_add], ones, mask=m_add)
            return carry2

        lax.fori_loop(0, CHUNK // 16, vec_body, 0)
        return carry

    lax.fori_loop(0, NCHUNK, chunk_body, 0)
    pltpu.sync_copy(histbuf, hist_hbm.at[wid])


def _sc_hist(keys3d, par):
    mesh = plsc.VectorSubcoreMesh(core_axis_name="c", subcore_axis_name="s")
    f = pl.kernel(
        _sc_hist_body,
        out_type=jax.ShapeDtypeStruct((NW, HIST), jnp.int32),
        mesh=mesh,
        compiler_params=pltpu.CompilerParams(needs_layout_passes=False),
        scratch_types=[
            pltpu.VMEM((CHUNK,), jnp.int32),
            pltpu.VMEM((64,), jnp.int32),
            pltpu.VMEM((HIST,), jnp.int32),
        ],
    )
    return f(keys3d, par)


def _search_body(h_ref, r_ref, o_ref, *, stage1):
    def hsum(w):
        def body(i, acc):
            return acc + h_ref[i]
        return lax.fori_loop(0, NW, body,
                             jnp.zeros((HIST // 128, 128), jnp.int32))

    H = hsum(None)
    flat = (lax.broadcasted_iota(jnp.int32, (HIST // 128, 128), 0) * 128
            + lax.broadcasted_iota(jnp.int32, (HIST // 128, 128), 1))
    binmap = flat >> 4

    def count_le(d, lo_bin, hi_bin):
        m = jnp.logical_and(binmap >= lo_bin, binmap <= jnp.minimum(d, hi_bin))
        return jnp.sum(jnp.where(m, H, 0))

    nnz_rm = count_le(NBIN - 1, 0, NBIN - 1)
    nnz_add = count_le(2 * NBIN - 1, NBIN, 2 * NBIN - 1)

    if stage1:
        nc = r_ref[0]
        n_rm = jnp.minimum(nnz_rm, nc)
        n_add = jnp.minimum(nnz_add, nc)
        r_rm = jnp.maximum(n_rm, 1)
        r_add = nnz_add - jnp.maximum(n_add, 1) + 1
    else:
        n_rm = jnp.int32(0)
        n_add = jnp.int32(0)
        r_rm = r_ref[0]
        r_add = r_ref[1]

    def bsearch(rank, lo_bin, hi_bin):
        def body(it, lohi):
            lo, hi = lohi
            mid = (lo + hi) >> 1
            c = count_le(mid, lo_bin, hi_bin)
            pred = c >= rank
            return (jnp.where(pred, lo, mid + 1), jnp.where(pred, mid, hi))

        lo, hi = lax.fori_loop(0, 10, body, (lo_bin, hi_bin))
        cb = count_le(lo - 1, lo_bin, hi_bin)
        return lo, cb

    p_rm, cb_rm = bsearch(r_rm, 0, NBIN - 1)
    p_add, cb_add = bsearch(r_add, NBIN, 2 * NBIN - 1)

    vals = [p_rm, r_rm - cb_rm, p_add - NBIN, r_add - cb_add,
            nnz_rm, nnz_add, n_rm, n_add]
    ridx = lax.broadcasted_iota(jnp.int32, (8, 128), 0)
    out = jnp.zeros((8, 128), jnp.int32)
    for i, v in enumerate(vals):
        out = jnp.where(ridx == i, v, out)
    o_ref[:] = out


def _search(hist, scal, stage1):
    return pl.pallas_call(
        functools.partial(_search_body, stage1=stage1),
        in_specs=[
            pl.BlockSpec(memory_space=pltpu.VMEM),
            pl.BlockSpec(memory_space=pltpu.SMEM),
        ],
        out_shape=jax.ShapeDtypeStruct((8, 128), jnp.int32),
    )(hist.reshape(NW, HIST // 128, 128), scal)


def _build_body(kA_ref, kB_ref, adj_ref, scal_ref, A_ref, rsabs_ref, rs_ref,
                acc_abs, acc):
    gi = pl.program_id(0)
    gj = pl.program_id(1)
    t_rm = scal_ref[0]
    t_add = scal_ref[1]
    rm_bin = scal_ref[2] > 0.0
    add_bin = scal_ref[3] > 0.0

    kA = kA_ref[:]
    kB = kB_ref[:].T
    adj = adj_ref[:]
    epA = lax.bitcast_convert_type(
        lax.bitwise_and(kA, jnp.int32(0x3FFFFFFF)), jnp.float32)
    epB = lax.bitcast_convert_type(
        lax.bitwise_and(kB, jnp.int32(0x3FFFFFFF)), jnp.float32)

    uA = jnp.logical_and(kA > 0, epA <= t_rm)
    uB = jnp.logical_and(kB > 0, epB <= t_rm)
    rm_b = jnp.logical_or(uA, uB).astype(jnp.float32)
    rm_raw = jnp.where(kA > 0, epA, 0.0)
    mask_rm = jnp.where(rm_bin, rm_b, rm_raw)

    aA = jnp.logical_and(kA < 0, epA >= t_add)
    aB = jnp.logical_and(kB < 0, epB >= t_add)
    add_b = jnp.logical_or(aA, aB).astype(jnp.float32)
    add_raw = jnp.where(kA < 0, epA, 0.0)
    mask_add = jnp.where(add_bin, add_b, add_raw)

    Ablk = adj - mask_rm + mask_add
    rows = gi * BLK + lax.broadcasted_iota(jnp.int32, (BLK, BLK), 0)
    cols = gj * BLK + lax.broadcasted_iota(jnp.int32, (BLK, BLK), 1)
    Ablk = jnp.where(rows == cols, 1.0, Ablk)
    A_ref[:] = Ablk

    @pl.when(gj == 0)
    def _():
        acc_abs[:] = jnp.zeros_like(acc_abs)
        acc[:] = jnp.zeros_like(acc)

    acc_abs[:] = acc_abs[:] + jnp.sum(jnp.abs(Ablk), axis=1, keepdims=True)
    acc[:] = acc[:] + jnp.sum(Ablk, axis=1, keepdims=True)

    @pl.when(gj == GR - 1)
    def _():
        rsabs_ref[:] = acc_abs[:]
        rs_ref[:] = acc[:]


def _build(keys, adj_ori, scal):
    return pl.pallas_call(
        _build_body,
        grid=(GR, GR),
        in_specs=[
            pl.BlockSpec((BLK, BLK), lambda i, j: (i, j)),
            pl.BlockSpec((BLK, BLK), lambda i, j: (j, i)),
            pl.BlockSpec((BLK, BLK), lambda i, j: (i, j)),
            pl.BlockSpec(memory_space=pltpu.SMEM),
        ],
        out_specs=[
            pl.BlockSpec((BLK, BLK), lambda i, j: (i, j)),
            pl.BlockSpec((BLK, 1), lambda i, j: (i, 0)),
            pl.BlockSpec((BLK, 1), lambda i, j: (i, 0)),
        ],
        out_shape=[
            jax.ShapeDtypeStruct((N, N), jnp.float32),
            jax.ShapeDtypeStruct((N, 1), jnp.float32),
            jax.ShapeDtypeStruct((N, 1), jnp.float32),
        ],
        scratch_shapes=[
            pltpu.VMEM((BLK, 1), jnp.float32),
            pltpu.VMEM((BLK, 1), jnp.float32),
        ],
    )(keys, keys, adj_ori, scal)


def _gcn1_body(A_ref, fj_ref, fi_ref, sc_ref, w1a_ref, w1b_ref, b1_ref,
               x_ref, acc):
    gj = pl.program_id(1)

    @pl.when(gj == 0)
    def _():
        acc[:] = jnp.zeros_like(acc)

    acc[:] = acc[:] + jnp.dot(A_ref[:], fj_ref[:],
                              preferred_element_type=jnp.float32)

    @pl.when(gj == GR - 1)
    def _():
        mean = acc[:] * sc_ref[:]
        x = (jnp.dot(fi_ref[:], w1a_ref[:], preferred_element_type=jnp.float32)
             + jnp.dot(mean, w1b_ref[:], preferred_element_type=jnp.float32)
             + b1_ref[:])
        x_ref[:] = jnp.maximum(x, 0.0)


def _gcn1(A, feats, scale, w1a, w1b, b1):
    return pl.pallas_call(
        _gcn1_body,
        grid=(GR, GR),
        in_specs=[
            pl.BlockSpec((BLK, BLK), lambda i, j: (i, j)),
            pl.BlockSpec((BLK, 256), lambda i, j: (j, 0)),
            pl.BlockSpec((BLK, 256), lambda i, j: (i, 0)),
            pl.BlockSpec((BLK, 1), lambda i, j: (i, 0)),
            pl.BlockSpec((256, 128), lambda i, j: (0, 0)),
            pl.BlockSpec((256, 128), lambda i, j: (0, 0)),
            pl.BlockSpec((1, 128), lambda i, j: (0, 0)),
        ],
        out_specs=pl.BlockSpec((BLK, 128), lambda i, j: (i, 0)),
        out_shape=jax.ShapeDtypeStruct((N, 128), jnp.float32),
        scratch_shapes=[pltpu.VMEM((BLK, 256), jnp.float32)],
    )(A, feats, feats, scale, w1a, w1b, b1)


def _gcn2_body(A_ref, xj_ref, xi_ref, sc_ref, w2a_ref, w2b_ref, b2_ref,
               csd_ref, img_ref, ft_ref, pt_ref, pi_ref, acc):
    gj = pl.program_id(1)

    @pl.when(gj == 0)
    def _():
        acc[:] = jnp.zeros_like(acc)

    acc[:] = acc[:] + jnp.dot(A_ref[:], xj_ref[:],
                              preferred_element_type=jnp.float32)

    @pl.when(gj == GR - 1)
    def _():
        mean2 = acc[:] * sc_ref[:]
        x2 = (jnp.dot(xi_ref[:], w2a_ref[:], preferred_element_type=jnp.float32)
              + jnp.dot(mean2, w2b_ref[:], preferred_element_type=jnp.float32)
              + b2_ref[:])
        m = jnp.max(x2, axis=1, keepdims=True)
        sh = x2 - m
        lse = jnp.log(jnp.sum(jnp.exp(sh), axis=1, keepdims=True))
        ft = sh - lse
        ft_ref[:] = ft
        pt_ref[:] = lax.dot_general(ft, csd_ref[:], (((1,), (1,)), ((), ())),
                                    preferred_element_type=jnp.float32)
        pi_ref[:] = lax.dot_general(img_ref[:], csd_ref[:],
                                    (((1,), (1,)), ((), ())),
                                    preferred_element_type=jnp.float32)


def _gcn2(A, x, scale, w2a, w2b, b2, csd_img, img_feats):
    return pl.pallas_call(
        _gcn2_body,
        grid=(GR, GR),
        in_specs=[
            pl.BlockSpec((BLK, BLK), lambda i, j: (i, j)),
            pl.BlockSpec((BLK, 128), lambda i, j: (j, 0)),
            pl.BlockSpec((BLK, 128), lambda i, j: (i, 0)),
            pl.BlockSpec((BLK, 1), lambda i, j: (i, 0)),
            pl.BlockSpec((128, 128), lambda i, j: (0, 0)),
            pl.BlockSpec((128, 128), lambda i, j: (0, 0)),
            pl.BlockSpec((1, 128), lambda i, j: (0, 0)),
            pl.BlockSpec((40, 128), lambda i, j: (0, 0)),
            pl.BlockSpec((BLK, 128), lambda i, j: (i, 0)),
        ],
        out_specs=[
            pl.BlockSpec((BLK, 128), lambda i, j: (i, 0)),
            pl.BlockSpec((BLK, 40), lambda i, j: (i, 0)),
            pl.BlockSpec((BLK, 40), lambda i, j: (i, 0)),
        ],
        out_shape=[
            jax.ShapeDtypeStruct((N, 128), jnp.float32),
            jax.ShapeDtypeStruct((N, 40), jnp.float32),
            jax.ShapeDtypeStruct((N, 40), jnp.float32),
        ],
        scratch_shapes=[pltpu.VMEM((BLK, 128), jnp.float32)],
    )(A, x, x, scale, w2a, w2b, b2, csd_img, img_feats)


def _splat16(v):
    return jnp.full((16,), v, jnp.int32)


def kernel(adj_norm, adj_ori, feats_ori, img_feats, csd_ori, csd_img,
           Wb, Wm, Wl, W1, b1, W2, b2):
    B1 = _mm(img_feats, Wb)
    hidden = _bigmm(adj_norm, B1)
    B2 = _mm(hidden, Wm)
    mean = _bigmm(adj_norm, B2, relu=True)
    adj_logits, mx = _logits(mean)
    maxv = jnp.max(mx).reshape(1)

    keys, cnt = _keys(adj_logits, adj_ori, maxv)
    n_edges = cnt[0, 0]
    n_change = n_edges // 2

    keys3d = keys.reshape(NW, NCHUNK, CHUNK)

    par1 = jnp.concatenate([_splat16(0), _splat16(0),
                            _splat16(30), _splat16(20)])
    h1 = _sc_hist(keys3d, par1)
    s1 = _search(h1, jnp.stack([n_change, n_change]), stage1=True)
    p1_rm, r2_rm, p1_add, r2_add = s1[0, 0], s1[1, 0], s1[2, 0], s1[3, 0]
    nnz_rm, nnz_add, n_rm, n_add = s1[4, 0], s1[5, 0], s1[6, 0], s1[7, 0]

    par2 = jnp.concatenate([_splat16(p1_rm), _splat16(p1_add),
                            _splat16(20), _splat16(10)])
    h2 = _sc_hist(keys3d, par2)
    s2 = _search(h2, jnp.stack([r2_rm, r2_add]), stage1=False)
    p2_rm, r3_rm, p2_add, r3_add = s2[0, 0], s2[1, 0], s2[2, 0], s2[3, 0]

    pre_rm = (p1_rm << 10) | p2_rm
    pre_add = (p1_add << 10) | p2_add
    par3 = jnp.concatenate([_splat16(pre_rm), _splat16(pre_add),
                            _splat16(10), _splat16(0)])
    h3 = _sc_hist(keys3d, par3)
    s3 = _search(h3, jnp.stack([r3_rm, r3_add]), stage1=False)
    p3_rm, p3_add = s3[0, 0], s3[2, 0]

    bits_rm = (pre_rm << 10) | p3_rm
    bits_add = (pre_add << 10) | p3_add
    thresh_rm = jnp.where(nnz_rm > 0,
                          lax.bitcast_convert_type(bits_rm, jnp.float32), 0.0)
    thresh_add = jnp.where(nnz_add > 0,
                           lax.bitcast_convert_type(bits_add, jnp.float32), 0.0)

    scal = jnp.stack([thresh_rm, thresh_add,
                      (n_rm > 0).astype(jnp.float32),
                      (n_add > 0).astype(jnp.float32)])

    A, rs_abs, rs = _build(keys, adj_ori, scal)
    denom = jnp.maximum(rs_abs, 1e-12)
    deg = rs / denom + 1e-7
    scale = 1.0 / (denom * deg)

    w1a, w1b = W1[:256], W1[256:]
    w2a, w2b = W2[:128], W2[128:]
    x = _gcn1(A, feats_ori, scale, w1a, w1b, b1.reshape(1, 128))
    feat_total, preds_total, preds_img = _gcn2(
        A, x, scale, w2a, w2b, b2.reshape(1, 128), csd_img, img_feats)

    return preds_total, feat_total, preds_img, adj_logits


# speedup vs baseline: 30.5316x; 1.1908x over previous
import functools

import jax
import jax.numpy as jnp
from jax import lax
from jax.experimental import pallas as pl
from jax.experimental.pallas import tpu as pltpu
from jax.experimental.pallas import tpu_sc as plsc

N = 4096
BLK = 256
GR = N // BLK

NW = 32
CHUNK = 8192
NCHUNK = N * N // NW // CHUNK
NBIN = 1024
SENT = 0x7FFFFFFF
NBINF = 4096
HIST = NBINF * 16


def _mm_body(a_ref, b_ref, o_ref):
    o_ref[:] = jnp.dot(a_ref[:], b_ref[:], preferred_element_type=jnp.float32)


def _mm(a, b):
    return pl.pallas_call(
        _mm_body,
        out_shape=jax.ShapeDtypeStruct((a.shape[0], b.shape[1]), jnp.float32),
    )(a, b)


def _bigmm_body(a_ref, b_ref, o_ref, *, relu):
    r = jnp.dot(a_ref[:], b_ref[:], preferred_element_type=jnp.float32)
    if relu:
        r = jnp.maximum(r, 0.0)
    o_ref[:] = r


def _bigmm(a, b, relu=False):
    k = b.shape[1]
    return pl.pallas_call(
        functools.partial(_bigmm_body, relu=relu),
        grid=(GR,),
        in_specs=[
            pl.BlockSpec((BLK, N), lambda i: (i, 0)),
            pl.BlockSpec((N, k), lambda i: (0, 0)),
        ],
        out_specs=pl.BlockSpec((BLK, k), lambda i: (i, 0)),
        out_shape=jax.ShapeDtypeStruct((N, k), jnp.float32),
    )(a, b)


def _logits_body(mi_ref, mj_ref, o_ref, mx_ref):
    gi = pl.program_id(0)
    gj = pl.program_id(1)
    lg = lax.dot_general(mi_ref[:], mj_ref[:], (((1,), (1,)), ((), ())),
                         preferred_element_type=jnp.float32)
    o_ref[:] = lg
    rows = gi * BLK + lax.broadcasted_iota(jnp.int32, (BLK, BLK), 0)
    cols = gj * BLK + lax.broadcasted_iota(jnp.int32, (BLK, BLK), 1)
    bm = jnp.max(jnp.where(cols > rows, lg, -jnp.inf))

    @pl.when(jnp.logical_and(gi == 0, gj == 0))
    def _():
        mx_ref[:] = jnp.full_like(mx_ref, -jnp.inf)

    mx_ref[:] = jnp.maximum(mx_ref[:], bm)


def _logits(mean):
    return pl.pallas_call(
        _logits_body,
        grid=(GR, GR),
        in_specs=[
            pl.BlockSpec((BLK, 64), lambda i, j: (i, 0)),
            pl.BlockSpec((BLK, 64), lambda i, j: (j, 0)),
        ],
        out_specs=[
            pl.BlockSpec((BLK, BLK), lambda i, j: (i, j)),
            pl.BlockSpec((8, 128), lambda i, j: (0, 0)),
        ],
        out_shape=[
            jax.ShapeDtypeStruct((N, N), jnp.float32),
            jax.ShapeDtypeStruct((8, 128), jnp.float32),
        ],
    )(mean, mean)


def _key_body(lg_ref, adj_ref, mx_ref, key_ref, cnt_ref):
    gi = pl.program_id(0)
    maxv = mx_ref[0]
    lg = lg_ref[:]
    adj = adj_ref[:]
    ep = lg / maxv
    rows = gi * BLK + lax.broadcasted_iota(jnp.int32, (BLK, N), 0)
    cols = lax.broadcasted_iota(jnp.int32, (BLK, N), 1)
    valid = jnp.logical_and(cols > rows, ep > 0.0)
    bits = lax.bitcast_convert_type(ep, jnp.int32)
    keyed = jnp.where(adj == 0.0, bits | jnp.int32(-2147483648), bits)
    key_ref[:] = jnp.where(valid, keyed, jnp.int32(SENT))
    cnt = jnp.sum((adj != 0.0).astype(jnp.int32))

    @pl.when(gi == 0)
    def _():
        cnt_ref[:] = jnp.zeros_like(cnt_ref)

    cnt_ref[:] = cnt_ref[:] + cnt


def _keys(logits, adj_ori, maxv):
    return pl.pallas_call(
        _key_body,
        grid=(GR,),
        in_specs=[
            pl.BlockSpec((BLK, N), lambda i: (i, 0)),
            pl.BlockSpec((BLK, N), lambda i: (i, 0)),
            pl.BlockSpec(memory_space=pltpu.SMEM),
        ],
        out_specs=[
            pl.BlockSpec((BLK, N), lambda i: (i, 0)),
            pl.BlockSpec((8, 128), lambda i: (0, 0)),
        ],
        out_shape=[
            jax.ShapeDtypeStruct((N, N), jnp.int32),
            jax.ShapeDtypeStruct((8, 128), jnp.int32),
        ],
    )(logits, adj_ori, maxv)


def _sc_hist_body(key_hbm, par_hbm, hist_hbm, keybuf, parbuf, histbuf,
                  sem0, sem1, *, masked):
    c = lax.axis_index("c")
    s = lax.axis_index("s")
    wid = s * 2 + c
    pltpu.sync_copy(par_hbm, parbuf)

    def zero_body(i, carry):
        histbuf[pl.ds(i * 16, 16)] = jnp.zeros((16,), jnp.int32)
        return carry

    lax.fori_loop(0, HIST // 16, zero_body, 0, unroll=8)

    p_rm = parbuf[pl.ds(0, 16)]
    p_add = parbuf[pl.ds(16, 16)]
    s_p = parbuf[pl.ds(32, 16)]
    s_d = parbuf[pl.ds(48, 16)]
    lane = lax.iota(jnp.int32, 16)
    ones = jnp.ones((16,), jnp.int32)

    def process(base):
        def vec_body(j, carry2):
            k = keybuf[pl.ds(base + j * 16, 16)]
            dig = lax.shift_right_logical(k, s_d)
            if masked:
                dig = lax.bitwise_and(dig, jnp.int32(NBIN - 1))
                sign = lax.shift_right_logical(k, 31)
                dig = lax.bitwise_or(dig, lax.shift_left(sign, 11))
            idx = lax.bitwise_or(lax.shift_left(dig, 4), lane)
            if masked:
                pref = lax.shift_right_logical(k, s_p)
                m = jnp.logical_or(pref == p_rm, pref == p_add)
                plsc.addupdate_scatter(histbuf, [idx], ones, mask=m)
            else:
                plsc.addupdate_scatter(histbuf, [idx], ones)
            return carry2

        lax.fori_loop(0, CHUNK // 16, vec_body, 0, unroll=8)

    def start(ci, half, sem):
        pltpu.make_async_copy(
            key_hbm.at[wid, ci],
            keybuf.at[pl.ds(half * CHUNK, CHUNK)], sem).start()

    def wait(half, sem):
        pltpu.make_async_copy(
            key_hbm.at[wid, 0],
            keybuf.at[pl.ds(half * CHUNK, CHUNK)], sem).wait()

    start(0, 0, sem0)

    def pair_body(i, carry):
        c0 = i * 2
        start(c0 + 1, 1, sem1)
        wait(0, sem0)
        process(0)

        @pl.when(c0 + 2 < NCHUNK)
        def _():
            start(c0 + 2, 0, sem0)

        wait(1, sem1)
        process(CHUNK)
        return carry

    lax.fori_loop(0, NCHUNK // 2, pair_body, 0)
    pltpu.sync_copy(histbuf, hist_hbm.at[wid])


def _sc_hist(keys3d, par, masked):
    mesh = plsc.VectorSubcoreMesh(core_axis_name="c", subcore_axis_name="s")
    f = pl.kernel(
        functools.partial(_sc_hist_body, masked=masked),
        out_type=jax.ShapeDtypeStruct((NW, HIST), jnp.int32),
        mesh=mesh,
        compiler_params=pltpu.CompilerParams(needs_layout_passes=False),
        scratch_types=[
            pltpu.VMEM((2 * CHUNK,), jnp.int32),
            pltpu.VMEM((64,), jnp.int32),
            pltpu.VMEM((HIST,), jnp.int32),
            pltpu.SemaphoreType.DMA,
            pltpu.SemaphoreType.DMA,
        ],
    )
    return f(keys3d, par)


def _search_body(h_ref, r_ref, o_ref, *, stage1):
    def hsum(w):
        def body(i, acc):
            return acc + h_ref[i]
        return lax.fori_loop(0, NW, body,
                             jnp.zeros((HIST // 128, 128), jnp.int32))

    H = hsum(None)
    flat = (lax.broadcasted_iota(jnp.int32, (HIST // 128, 128), 0) * 128
            + lax.broadcasted_iota(jnp.int32, (HIST // 128, 128), 1))
    binmap = flat >> 4

    def count_le(d, lo_bin, hi_bin):
        m = jnp.logical_and(binmap >= lo_bin, binmap <= jnp.minimum(d, hi_bin))
        return jnp.sum(jnp.where(m, H, 0))

    RM_HI = 2 * NBIN - 2
    ADD_LO = 2 * NBIN
    ADD_HI = 4 * NBIN - 1
    nnz_rm = count_le(RM_HI, 0, RM_HI)
    nnz_add = count_le(ADD_HI, ADD_LO, ADD_HI)

    if stage1:
        nc = r_ref[0]
        n_rm = jnp.minimum(nnz_rm, nc)
        n_add = jnp.minimum(nnz_add, nc)
        r_rm = jnp.maximum(n_rm, 1)
        r_add = nnz_add - jnp.maximum(n_add, 1) + 1
    else:
        n_rm = jnp.int32(0)
        n_add = jnp.int32(0)
        r_rm = r_ref[0]
        r_add = r_ref[1]

    def bsearch(rank, lo_bin, hi_bin):
        def body(it, lohi):
            lo, hi = lohi
            mid = (lo + hi) >> 1
            c = count_le(mid, lo_bin, hi_bin)
            pred = c >= rank
            return (jnp.where(pred, lo, mid + 1), jnp.where(pred, mid, hi))

        lo, hi = lax.fori_loop(0, 11, body, (lo_bin, hi_bin))
        cb = count_le(lo - 1, lo_bin, hi_bin)
        return lo, cb

    p_rm, cb_rm = bsearch(r_rm, 0, RM_HI)
    p_add, cb_add = bsearch(r_add, ADD_LO, ADD_HI)

    vals = [p_rm, r_rm - cb_rm, p_add - ADD_LO, r_add - cb_add,
            nnz_rm, nnz_add, n_rm, n_add]
    ridx = lax.broadcasted_iota(jnp.int32, (8, 128), 0)
    out = jnp.zeros((8, 128), jnp.int32)
    for i, v in enumerate(vals):
        out = jnp.where(ridx == i, v, out)
    o_ref[:] = out


def _search(hist, scal, stage1):
    return pl.pallas_call(
        functools.partial(_search_body, stage1=stage1),
        in_specs=[
            pl.BlockSpec(memory_space=pltpu.VMEM),
            pl.BlockSpec(memory_space=pltpu.SMEM),
        ],
        out_shape=jax.ShapeDtypeStruct((8, 128), jnp.int32),
    )(hist.reshape(NW, HIST // 128, 128), scal)


def _build_body(kA_ref, kB_ref, adj_ref, scal_ref, A_ref, rsabs_ref, rs_ref,
                acc_abs, acc):
    gi = pl.program_id(0)
    gj = pl.program_id(1)
    t_rm = scal_ref[0]
    t_add = scal_ref[1]
    rm_bin = scal_ref[2] > 0.0
    add_bin = scal_ref[3] > 0.0

    kA = kA_ref[:]
    kB = kB_ref[:].T
    adj = adj_ref[:]
    epA = lax.bitcast_convert_type(
        lax.bitwise_and(kA, jnp.int32(0x3FFFFFFF)), jnp.float32)
    epB = lax.bitcast_convert_type(
        lax.bitwise_and(kB, jnp.int32(0x3FFFFFFF)), jnp.float32)

    vA = jnp.logical_and(kA > 0, kA < jnp.int32(0x40000000))
    vB = jnp.logical_and(kB > 0, kB < jnp.int32(0x40000000))
    uA = jnp.logical_and(vA, epA <= t_rm)
    uB = jnp.logical_and(vB, epB <= t_rm)
    rm_b = jnp.logical_or(uA, uB).astype(jnp.float32)
    rm_raw = jnp.where(vA, epA, 0.0)
    mask_rm = jnp.where(rm_bin, rm_b, rm_raw)

    aA = jnp.logical_and(kA < 0, epA >= t_add)
    aB = jnp.logical_and(kB < 0, epB >= t_add)
    add_b = jnp.logical_or(aA, aB).astype(jnp.float32)
    add_raw = jnp.where(kA < 0, epA, 0.0)
    mask_add = jnp.where(add_bin, add_b, add_raw)

    Ablk = adj - mask_rm + mask_add
    rows = gi * BLK + lax.broadcasted_iota(jnp.int32, (BLK, BLK), 0)
    cols = gj * BLK + lax.broadcasted_iota(jnp.int32, (BLK, BLK), 1)
    Ablk = jnp.where(rows == cols, 1.0, Ablk)
    A_ref[:] = Ablk

    @pl.when(gj == 0)
    def _():
        acc_abs[:] = jnp.zeros_like(acc_abs)
        acc[:] = jnp.zeros_like(acc)

    acc_abs[:] = acc_abs[:] + jnp.sum(jnp.abs(Ablk), axis=1, keepdims=True)
    acc[:] = acc[:] + jnp.sum(Ablk, axis=1, keepdims=True)

    @pl.when(gj == GR - 1)
    def _():
        rsabs_ref[:] = acc_abs[:]
        rs_ref[:] = acc[:]


def _build(keys, adj_ori, scal):
    return pl.pallas_call(
        _build_body,
        grid=(GR, GR),
        in_specs=[
            pl.BlockSpec((BLK, BLK), lambda i, j: (i, j)),
            pl.BlockSpec((BLK, BLK), lambda i, j: (j, i)),
            pl.BlockSpec((BLK, BLK), lambda i, j: (i, j)),
            pl.BlockSpec(memory_space=pltpu.SMEM),
        ],
        out_specs=[
            pl.BlockSpec((BLK, BLK), lambda i, j: (i, j)),
            pl.BlockSpec((BLK, 1), lambda i, j: (i, 0)),
            pl.BlockSpec((BLK, 1), lambda i, j: (i, 0)),
        ],
        out_shape=[
            jax.ShapeDtypeStruct((N, N), jnp.float32),
            jax.ShapeDtypeStruct((N, 1), jnp.float32),
            jax.ShapeDtypeStruct((N, 1), jnp.float32),
        ],
        scratch_shapes=[
            pltpu.VMEM((BLK, 1), jnp.float32),
            pltpu.VMEM((BLK, 1), jnp.float32),
        ],
    )(keys, keys, adj_ori, scal)


def _gcn1_body(A_ref, fj_ref, fi_ref, sc_ref, w1a_ref, w1b_ref, b1_ref,
               x_ref, acc):
    gj = pl.program_id(1)

    @pl.when(gj == 0)
    def _():
        acc[:] = jnp.zeros_like(acc)

    acc[:] = acc[:] + jnp.dot(A_ref[:], fj_ref[:],
                              preferred_element_type=jnp.float32)

    @pl.when(gj == GR - 1)
    def _():
        mean = acc[:] * sc_ref[:]
        x = (jnp.dot(fi_ref[:], w1a_ref[:], preferred_element_type=jnp.float32)
             + jnp.dot(mean, w1b_ref[:], preferred_element_type=jnp.float32)
             + b1_ref[:])
        x_ref[:] = jnp.maximum(x, 0.0)


def _gcn1(A, feats, scale, w1a, w1b, b1):
    return pl.pallas_call(
        _gcn1_body,
        grid=(GR, GR),
        in_specs=[
            pl.BlockSpec((BLK, BLK), lambda i, j: (i, j)),
            pl.BlockSpec((BLK, 256), lambda i, j: (j, 0)),
            pl.BlockSpec((BLK, 256), lambda i, j: (i, 0)),
            pl.BlockSpec((BLK, 1), lambda i, j: (i, 0)),
            pl.BlockSpec((256, 128), lambda i, j: (0, 0)),
            pl.BlockSpec((256, 128), lambda i, j: (0, 0)),
            pl.BlockSpec((1, 128), lambda i, j: (0, 0)),
        ],
        out_specs=pl.BlockSpec((BLK, 128), lambda i, j: (i, 0)),
        out_shape=jax.ShapeDtypeStruct((N, 128), jnp.float32),
        scratch_shapes=[pltpu.VMEM((BLK, 256), jnp.float32)],
    )(A, feats, feats, scale, w1a, w1b, b1)


def _gcn2_body(A_ref, xj_ref, xi_ref, sc_ref, w2a_ref, w2b_ref, b2_ref,
               csd_ref, img_ref, ft_ref, pt_ref, pi_ref, acc):
    gj = pl.program_id(1)

    @pl.when(gj == 0)
    def _():
        acc[:] = jnp.zeros_like(acc)

    acc[:] = acc[:] + jnp.dot(A_ref[:], xj_ref[:],
                              preferred_element_type=jnp.float32)

    @pl.when(gj == GR - 1)
    def _():
        mean2 = acc[:] * sc_ref[:]
        x2 = (jnp.dot(xi_ref[:], w2a_ref[:], preferred_element_type=jnp.float32)
              + jnp.dot(mean2, w2b_ref[:], preferred_element_type=jnp.float32)
              + b2_ref[:])
        m = jnp.max(x2, axis=1, keepdims=True)
        sh = x2 - m
        lse = jnp.log(jnp.sum(jnp.exp(sh), axis=1, keepdims=True))
        ft = sh - lse
        ft_ref[:] = ft
        pt_ref[:] = lax.dot_general(ft, csd_ref[:], (((1,), (1,)), ((), ())),
                                    preferred_element_type=jnp.float32)
        pi_ref[:] = lax.dot_general(img_ref[:], csd_ref[:],
                                    (((1,), (1,)), ((), ())),
                                    preferred_element_type=jnp.float32)


def _gcn2(A, x, scale, w2a, w2b, b2, csd_img, img_feats):
    return pl.pallas_call(
        _gcn2_body,
        grid=(GR, GR),
        in_specs=[
            pl.BlockSpec((BLK, BLK), lambda i, j: (i, j)),
            pl.BlockSpec((BLK, 128), lambda i, j: (j, 0)),
            pl.BlockSpec((BLK, 128), lambda i, j: (i, 0)),
            pl.BlockSpec((BLK, 1), lambda i, j: (i, 0)),
            pl.BlockSpec((128, 128), lambda i, j: (0, 0)),
            pl.BlockSpec((128, 128), lambda i, j: (0, 0)),
            pl.BlockSpec((1, 128), lambda i, j: (0, 0)),
            pl.BlockSpec((40, 128), lambda i, j: (0, 0)),
            pl.BlockSpec((BLK, 128), lambda i, j: (i, 0)),
        ],
        out_specs=[
            pl.BlockSpec((BLK, 128), lambda i, j: (i, 0)),
            pl.BlockSpec((BLK, 40), lambda i, j: (i, 0)),
            pl.BlockSpec((BLK, 40), lambda i, j: (i, 0)),
        ],
        out_shape=[
            jax.ShapeDtypeStruct((N, 128), jnp.float32),
            jax.ShapeDtypeStruct((N, 40), jnp.float32),
            jax.ShapeDtypeStruct((N, 40), jnp.float32),
        ],
        scratch_shapes=[pltpu.VMEM((BLK, 128), jnp.float32)],
    )(A, x, x, scale, w2a, w2b, b2, csd_img, img_feats)


def _splat16(v):
    return jnp.full((16,), v, jnp.int32)


def kernel(adj_norm, adj_ori, feats_ori, img_feats, csd_ori, csd_img,
           Wb, Wm, Wl, W1, b1, W2, b2):
    B1 = _mm(img_feats, Wb)
    hidden = _bigmm(adj_norm, B1)
    B2 = _mm(hidden, Wm)
    mean = _bigmm(adj_norm, B2, relu=True)
    adj_logits, mx = _logits(mean)
    maxv = jnp.max(mx).reshape(1)

    keys, cnt = _keys(adj_logits, adj_ori, maxv)
    n_edges = cnt[0, 0]
    n_change = n_edges // 2

    keys3d = keys.reshape(NW, NCHUNK, CHUNK)

    par1 = jnp.concatenate([_splat16(0), _splat16(0),
                            _splat16(31), _splat16(20)])
    h1 = _sc_hist(keys3d, par1, masked=False)
    s1 = _search(h1, jnp.stack([n_change, n_change]), stage1=True)
    p1_rm, r2_rm, p1_add, r2_add = s1[0, 0], s1[1, 0], s1[2, 0], s1[3, 0]
    nnz_rm, nnz_add, n_rm, n_add = s1[4, 0], s1[5, 0], s1[6, 0], s1[7, 0]

    par2 = jnp.concatenate([_splat16(p1_rm), _splat16(2048 + p1_add),
                            _splat16(20), _splat16(10)])
    h2 = _sc_hist(keys3d, par2, masked=True)
    s2 = _search(h2, jnp.stack([r2_rm, r2_add]), stage1=False)
    p2_rm, r3_rm, p2_add, r3_add = s2[0, 0], s2[1, 0], s2[2, 0], s2[3, 0]

    pre_rm = (p1_rm << 10) | p2_rm
    pre_add = (p1_add << 10) | p2_add
    par3 = jnp.concatenate([_splat16(pre_rm), _splat16((1 << 21) | pre_add),
                            _splat16(10), _splat16(0)])
    h3 = _sc_hist(keys3d, par3, masked=True)
    s3 = _search(h3, jnp.stack([r3_rm, r3_add]), stage1=False)
    p3_rm, p3_add = s3[0, 0], s3[2, 0]

    bits_rm = (pre_rm << 10) | p3_rm
    bits_add = (pre_add << 10) | p3_add
    thresh_rm = jnp.where(nnz_rm > 0,
                          lax.bitcast_convert_type(bits_rm, jnp.float32), 0.0)
    thresh_add = jnp.where(nnz_add > 0,
                           lax.bitcast_convert_type(bits_add, jnp.float32), 0.0)

    scal = jnp.stack([thresh_rm, thresh_add,
                      (n_rm > 0).astype(jnp.float32),
                      (n_add > 0).astype(jnp.float32)])

    A, rs_abs, rs = _build(keys, adj_ori, scal)
    denom = jnp.maximum(rs_abs, 1e-12)
    deg = rs / denom + 1e-7
    scale = 1.0 / (denom * deg)

    w1a, w1b = W1[:256], W1[256:]
    w2a, w2b = W2[:128], W2[128:]
    x = _gcn1(A, feats_ori, scale, w1a, w1b, b1.reshape(1, 128))
    feat_total, preds_total, preds_img = _gcn2(
        A, x, scale, w2a, w2b, b2.reshape(1, 128), csd_img, img_feats)

    return preds_total, feat_total, preds_img, adj_logits


# speedup vs baseline: 39.7879x; 1.3032x over previous
import functools

import jax
import jax.numpy as jnp
from jax import lax
from jax.experimental import pallas as pl
from jax.experimental.pallas import tpu as pltpu
from jax.experimental.pallas import tpu_sc as plsc

N = 4096
BLK = 256
GR = N // BLK
BB = 512
GR2 = N // BB

NW = 32
CHUNK = 8192
NCHUNK = N * N // NW // CHUNK
NBIN = 1024
SENT = 0x7FFFFFFF
NBINF = 4096
HIST = NBINF * 16


def _mm_body(a_ref, b_ref, o_ref):
    o_ref[:] = jnp.dot(a_ref[:], b_ref[:], preferred_element_type=jnp.float32)


def _mm(a, b):
    return pl.pallas_call(
        _mm_body,
        out_shape=jax.ShapeDtypeStruct((a.shape[0], b.shape[1]), jnp.float32),
    )(a, b)


def _bigmm_body(a_ref, b_ref, o_ref, *, relu):
    r = jnp.dot(a_ref[:], b_ref[:], preferred_element_type=jnp.float32)
    if relu:
        r = jnp.maximum(r, 0.0)
    o_ref[:] = r


def _bigmm(a, b, relu=False):
    k = b.shape[1]
    return pl.pallas_call(
        functools.partial(_bigmm_body, relu=relu),
        grid=(GR2,),
        in_specs=[
            pl.BlockSpec((BB, N), lambda i: (i, 0)),
            pl.BlockSpec((N, k), lambda i: (0, 0)),
        ],
        out_specs=pl.BlockSpec((BB, k), lambda i: (i, 0)),
        out_shape=jax.ShapeDtypeStruct((N, k), jnp.float32),
    )(a, b)


def _logits_body(mi_ref, mj_ref, o_ref, mx_ref):
    gi = pl.program_id(0)
    gj = pl.program_id(1)
    lg = lax.dot_general(mi_ref[:], mj_ref[:], (((1,), (1,)), ((), ())),
                         preferred_element_type=jnp.float32)
    o_ref[:] = lg
    rows = gi * BB + lax.broadcasted_iota(jnp.int32, (BB, BB), 0)
    cols = gj * BB + lax.broadcasted_iota(jnp.int32, (BB, BB), 1)
    bm = jnp.max(jnp.where(cols > rows, lg, -jnp.inf))

    @pl.when(jnp.logical_and(gi == 0, gj == 0))
    def _():
        mx_ref[:] = jnp.full_like(mx_ref, -jnp.inf)

    mx_ref[:] = jnp.maximum(mx_ref[:], bm)


def _logits(mean):
    return pl.pallas_call(
        _logits_body,
        grid=(GR2, GR2),
        in_specs=[
            pl.BlockSpec((BB, 64), lambda i, j: (i, 0)),
            pl.BlockSpec((BB, 64), lambda i, j: (j, 0)),
        ],
        out_specs=[
            pl.BlockSpec((BB, BB), lambda i, j: (i, j)),
            pl.BlockSpec((8, 128), lambda i, j: (0, 0)),
        ],
        out_shape=[
            jax.ShapeDtypeStruct((N, N), jnp.float32),
            jax.ShapeDtypeStruct((8, 128), jnp.float32),
        ],
    )(mean, mean)


def _key_body(lg_ref, adj_ref, mx_ref, key_ref, cnt_ref):
    gi = pl.program_id(0)
    maxv = mx_ref[0]
    lg = lg_ref[:]
    adj = adj_ref[:]
    ep = lg / maxv
    rows = gi * BLK + lax.broadcasted_iota(jnp.int32, (BLK, N), 0)
    cols = lax.broadcasted_iota(jnp.int32, (BLK, N), 1)
    valid = jnp.logical_and(cols > rows, ep > 0.0)
    bits = lax.bitcast_convert_type(ep, jnp.int32)
    keyed = jnp.where(adj == 0.0, bits | jnp.int32(-2147483648), bits)
    key_ref[:] = jnp.where(valid, keyed, jnp.int32(SENT))
    cnt = jnp.sum((adj != 0.0).astype(jnp.int32))

    @pl.when(gi == 0)
    def _():
        cnt_ref[:] = jnp.zeros_like(cnt_ref)

    cnt_ref[:] = cnt_ref[:] + cnt


def _keys(logits, adj_ori, maxv):
    return pl.pallas_call(
        _key_body,
        grid=(GR,),
        in_specs=[
            pl.BlockSpec((BLK, N), lambda i: (i, 0)),
            pl.BlockSpec((BLK, N), lambda i: (i, 0)),
            pl.BlockSpec(memory_space=pltpu.SMEM),
        ],
        out_specs=[
            pl.BlockSpec((BLK, N), lambda i: (i, 0)),
            pl.BlockSpec((8, 128), lambda i: (0, 0)),
        ],
        out_shape=[
            jax.ShapeDtypeStruct((N, N), jnp.int32),
            jax.ShapeDtypeStruct((8, 128), jnp.int32),
        ],
    )(logits, adj_ori, maxv)


def _sc_hist_body(key_hbm, par_hbm, hist_hbm, keybuf, parbuf, histbuf,
                  sem0, sem1, *, masked):
    c = lax.axis_index("c")
    s = lax.axis_index("s")
    wid = s * 2 + c
    pltpu.sync_copy(par_hbm, parbuf)

    def zero_body(i, carry):
        histbuf[pl.ds(i * 16, 16)] = jnp.zeros((16,), jnp.int32)
        return carry

    lax.fori_loop(0, HIST // 16, zero_body, 0, unroll=8)

    p_rm = parbuf[pl.ds(0, 16)]
    p_add = parbuf[pl.ds(16, 16)]
    s_p = parbuf[pl.ds(32, 16)]
    s_d = parbuf[pl.ds(48, 16)]
    lane = lax.iota(jnp.int32, 16)
    ones = jnp.ones((16,), jnp.int32)

    def process(base):
        def vec_body(j, carry2):
            k = keybuf[pl.ds(base + j * 16, 16)]
            dig = lax.shift_right_logical(k, s_d)
            if masked:
                dig = lax.bitwise_and(dig, jnp.int32(NBIN - 1))
                sign = lax.shift_right_logical(k, 31)
                dig = lax.bitwise_or(dig, lax.shift_left(sign, 11))
            idx = lax.bitwise_or(lax.shift_left(dig, 4), lane)
            if masked:
                pref = lax.shift_right_logical(k, s_p)
                m = jnp.logical_or(pref == p_rm, pref == p_add)
                plsc.addupdate_scatter(histbuf, [idx], ones, mask=m)
            else:
                plsc.addupdate_scatter(histbuf, [idx], ones)
            return carry2

        lax.fori_loop(0, CHUNK // 16, vec_body, 0, unroll=8)

    def start(ci, half, sem):
        pltpu.make_async_copy(
            key_hbm.at[wid, ci],
            keybuf.at[pl.ds(half * CHUNK, CHUNK)], sem).start()

    def wait(half, sem):
        pltpu.make_async_copy(
            key_hbm.at[wid, 0],
            keybuf.at[pl.ds(half * CHUNK, CHUNK)], sem).wait()

    start(0, 0, sem0)

    def pair_body(i, carry):
        c0 = i * 2
        start(c0 + 1, 1, sem1)
        wait(0, sem0)
        process(0)

        @pl.when(c0 + 2 < NCHUNK)
        def _():
            start(c0 + 2, 0, sem0)

        wait(1, sem1)
        process(CHUNK)
        return carry

    lax.fori_loop(0, NCHUNK // 2, pair_body, 0)
    pltpu.sync_copy(histbuf, hist_hbm.at[wid])


def _sc_hist(keys3d, par, masked):
    mesh = plsc.VectorSubcoreMesh(core_axis_name="c", subcore_axis_name="s")
    f = pl.kernel(
        functools.partial(_sc_hist_body, masked=masked),
        out_type=jax.ShapeDtypeStruct((NW, HIST), jnp.int32),
        mesh=mesh,
        compiler_params=pltpu.CompilerParams(needs_layout_passes=False),
        scratch_types=[
            pltpu.VMEM((2 * CHUNK,), jnp.int32),
            pltpu.VMEM((64,), jnp.int32),
            pltpu.VMEM((HIST,), jnp.int32),
            pltpu.SemaphoreType.DMA,
            pltpu.SemaphoreType.DMA,
        ],
    )
    return f(keys3d, par)


def _search_body(h_ref, r_ref, o_ref, *, stage1):
    def hsum(w):
        def body(i, acc):
            return acc + h_ref[i]
        return lax.fori_loop(0, NW, body,
                             jnp.zeros((HIST // 128, 128), jnp.int32))

    H = hsum(None)
    flat = (lax.broadcasted_iota(jnp.int32, (HIST // 128, 128), 0) * 128
            + lax.broadcasted_iota(jnp.int32, (HIST // 128, 128), 1))
    binmap = flat >> 4

    def count_le(d, lo_bin, hi_bin):
        m = jnp.logical_and(binmap >= lo_bin, binmap <= jnp.minimum(d, hi_bin))
        return jnp.sum(jnp.where(m, H, 0))

    RM_HI = 2 * NBIN - 2
    ADD_LO = 2 * NBIN
    ADD_HI = 4 * NBIN - 1
    nnz_rm = count_le(RM_HI, 0, RM_HI)
    nnz_add = count_le(ADD_HI, ADD_LO, ADD_HI)

    if stage1:
        nc = r_ref[0]
        n_rm = jnp.minimum(nnz_rm, nc)
        n_add = jnp.minimum(nnz_add, nc)
        r_rm = jnp.maximum(n_rm, 1)
        r_add = nnz_add - jnp.maximum(n_add, 1) + 1
    else:
        n_rm = jnp.int32(0)
        n_add = jnp.int32(0)
        r_rm = r_ref[0]
        r_add = r_ref[1]

    def bsearch(rank, lo_bin, hi_bin):
        def body(it, lohi):
            lo, hi = lohi
            mid = (lo + hi) >> 1
            c = count_le(mid, lo_bin, hi_bin)
            pred = c >= rank
            return (jnp.where(pred, lo, mid + 1), jnp.where(pred, mid, hi))

        lo, hi = lax.fori_loop(0, 11, body, (lo_bin, hi_bin))
        cb = count_le(lo - 1, lo_bin, hi_bin)
        return lo, cb

    p_rm, cb_rm = bsearch(r_rm, 0, RM_HI)
    p_add, cb_add = bsearch(r_add, ADD_LO, ADD_HI)

    vals = [p_rm, r_rm - cb_rm, p_add - ADD_LO, r_add - cb_add,
            nnz_rm, nnz_add, n_rm, n_add]
    ridx = lax.broadcasted_iota(jnp.int32, (8, 128), 0)
    out = jnp.zeros((8, 128), jnp.int32)
    for i, v in enumerate(vals):
        out = jnp.where(ridx == i, v, out)
    o_ref[:] = out


def _search(hist, scal, stage1):
    return pl.pallas_call(
        functools.partial(_search_body, stage1=stage1),
        in_specs=[
            pl.BlockSpec(memory_space=pltpu.VMEM),
            pl.BlockSpec(memory_space=pltpu.SMEM),
        ],
        out_shape=jax.ShapeDtypeStruct((8, 128), jnp.int32),
    )(hist.reshape(NW, HIST // 128, 128), scal)


def _build_body(kA_ref, kB_ref, adj_ref, scal_ref, A_ref, rsabs_ref, rs_ref,
                acc_abs, acc):
    gi = pl.program_id(0)
    gj = pl.program_id(1)
    t_rm = scal_ref[0]
    t_add = scal_ref[1]
    rm_bin = scal_ref[2] > 0.0
    add_bin = scal_ref[3] > 0.0

    kA = kA_ref[:]
    kB = kB_ref[:].T
    adj = adj_ref[:]
    epA = lax.bitcast_convert_type(
        lax.bitwise_and(kA, jnp.int32(0x3FFFFFFF)), jnp.float32)
    epB = lax.bitcast_convert_type(
        lax.bitwise_and(kB, jnp.int32(0x3FFFFFFF)), jnp.float32)

    vA = jnp.logical_and(kA > 0, kA < jnp.int32(0x40000000))
    vB = jnp.logical_and(kB > 0, kB < jnp.int32(0x40000000))
    uA = jnp.logical_and(vA, epA <= t_rm)
    uB = jnp.logical_and(vB, epB <= t_rm)
    rm_b = jnp.logical_or(uA, uB).astype(jnp.float32)
    rm_raw = jnp.where(vA, epA, 0.0)
    mask_rm = jnp.where(rm_bin, rm_b, rm_raw)

    aA = jnp.logical_and(kA < 0, epA >= t_add)
    aB = jnp.logical_and(kB < 0, epB >= t_add)
    add_b = jnp.logical_or(aA, aB).astype(jnp.float32)
    add_raw = jnp.where(kA < 0, epA, 0.0)
    mask_add = jnp.where(add_bin, add_b, add_raw)

    Ablk = adj - mask_rm + mask_add
    rows = gi * BB + lax.broadcasted_iota(jnp.int32, (BB, BB), 0)
    cols = gj * BB + lax.broadcasted_iota(jnp.int32, (BB, BB), 1)
    Ablk = jnp.where(rows == cols, 1.0, Ablk)
    A_ref[:] = Ablk

    @pl.when(gj == 0)
    def _():
        acc_abs[:] = jnp.zeros_like(acc_abs)
        acc[:] = jnp.zeros_like(acc)

    acc_abs[:] = acc_abs[:] + jnp.sum(jnp.abs(Ablk), axis=1, keepdims=True)
    acc[:] = acc[:] + jnp.sum(Ablk, axis=1, keepdims=True)

    @pl.when(gj == GR2 - 1)
    def _():
        rsabs_ref[:] = acc_abs[:]
        rs_ref[:] = acc[:]


def _build(keys, adj_ori, scal):
    return pl.pallas_call(
        _build_body,
        grid=(GR2, GR2),
        in_specs=[
            pl.BlockSpec((BB, BB), lambda i, j: (i, j)),
            pl.BlockSpec((BB, BB), lambda i, j: (j, i)),
            pl.BlockSpec((BB, BB), lambda i, j: (i, j)),
            pl.BlockSpec(memory_space=pltpu.SMEM),
        ],
        out_specs=[
            pl.BlockSpec((BB, BB), lambda i, j: (i, j)),
            pl.BlockSpec((BB, 1), lambda i, j: (i, 0)),
            pl.BlockSpec((BB, 1), lambda i, j: (i, 0)),
        ],
        out_shape=[
            jax.ShapeDtypeStruct((N, N), jnp.float32),
            jax.ShapeDtypeStruct((N, 1), jnp.float32),
            jax.ShapeDtypeStruct((N, 1), jnp.float32),
        ],
        scratch_shapes=[
            pltpu.VMEM((BB, 1), jnp.float32),
            pltpu.VMEM((BB, 1), jnp.float32),
        ],
    )(keys, keys, adj_ori, scal)


def _gcn1_body(A_ref, fj_ref, fi_ref, sc_ref, w1a_ref, w1b_ref, b1_ref,
               x_ref, acc):
    gj = pl.program_id(1)

    @pl.when(gj == 0)
    def _():
        acc[:] = jnp.zeros_like(acc)

    acc[:] = acc[:] + jnp.dot(A_ref[:], fj_ref[:],
                              preferred_element_type=jnp.float32)

    @pl.when(gj == GR2 - 1)
    def _():
        mean = acc[:] * sc_ref[:]
        x = (jnp.dot(fi_ref[:], w1a_ref[:], preferred_element_type=jnp.float32)
             + jnp.dot(mean, w1b_ref[:], preferred_element_type=jnp.float32)
             + b1_ref[:])
        x_ref[:] = jnp.maximum(x, 0.0)


def _gcn1(A, feats, scale, w1a, w1b, b1):
    return pl.pallas_call(
        _gcn1_body,
        grid=(GR2, GR2),
        in_specs=[
            pl.BlockSpec((BB, BB), lambda i, j: (i, j)),
            pl.BlockSpec((BB, 256), lambda i, j: (j, 0)),
            pl.BlockSpec((BB, 256), lambda i, j: (i, 0)),
            pl.BlockSpec((BB, 1), lambda i, j: (i, 0)),
            pl.BlockSpec((256, 128), lambda i, j: (0, 0)),
            pl.BlockSpec((256, 128), lambda i, j: (0, 0)),
            pl.BlockSpec((1, 128), lambda i, j: (0, 0)),
        ],
        out_specs=pl.BlockSpec((BB, 128), lambda i, j: (i, 0)),
        out_shape=jax.ShapeDtypeStruct((N, 128), jnp.float32),
        scratch_shapes=[pltpu.VMEM((BB, 256), jnp.float32)],
    )(A, feats, feats, scale, w1a, w1b, b1)


def _gcn2_body(A_ref, xj_ref, xi_ref, sc_ref, w2a_ref, w2b_ref, b2_ref,
               csd_ref, img_ref, ft_ref, pt_ref, pi_ref, acc):
    gj = pl.program_id(1)

    @pl.when(gj == 0)
    def _():
        acc[:] = jnp.zeros_like(acc)

    acc[:] = acc[:] + jnp.dot(A_ref[:], xj_ref[:],
                              preferred_element_type=jnp.float32)

    @pl.when(gj == GR2 - 1)
    def _():
        mean2 = acc[:] * sc_ref[:]
        x2 = (jnp.dot(xi_ref[:], w2a_ref[:], preferred_element_type=jnp.float32)
              + jnp.dot(mean2, w2b_ref[:], preferred_element_type=jnp.float32)
              + b2_ref[:])
        m = jnp.max(x2, axis=1, keepdims=True)
        sh = x2 - m
        lse = jnp.log(jnp.sum(jnp.exp(sh), axis=1, keepdims=True))
        ft = sh - lse
        ft_ref[:] = ft
        pt_ref[:] = lax.dot_general(ft, csd_ref[:], (((1,), (1,)), ((), ())),
                                    preferred_element_type=jnp.float32)
        pi_ref[:] = lax.dot_general(img_ref[:], csd_ref[:],
                                    (((1,), (1,)), ((), ())),
                                    preferred_element_type=jnp.float32)


def _gcn2(A, x, scale, w2a, w2b, b2, csd_img, img_feats):
    return pl.pallas_call(
        _gcn2_body,
        grid=(GR2, GR2),
        in_specs=[
            pl.BlockSpec((BB, BB), lambda i, j: (i, j)),
            pl.BlockSpec((BB, 128), lambda i, j: (j, 0)),
            pl.BlockSpec((BB, 128), lambda i, j: (i, 0)),
            pl.BlockSpec((BB, 1), lambda i, j: (i, 0)),
            pl.BlockSpec((128, 128), lambda i, j: (0, 0)),
            pl.BlockSpec((128, 128), lambda i, j: (0, 0)),
            pl.BlockSpec((1, 128), lambda i, j: (0, 0)),
            pl.BlockSpec((40, 128), lambda i, j: (0, 0)),
            pl.BlockSpec((BB, 128), lambda i, j: (i, 0)),
        ],
        out_specs=[
            pl.BlockSpec((BB, 128), lambda i, j: (i, 0)),
            pl.BlockSpec((BB, 40), lambda i, j: (i, 0)),
            pl.BlockSpec((BB, 40), lambda i, j: (i, 0)),
        ],
        out_shape=[
            jax.ShapeDtypeStruct((N, 128), jnp.float32),
            jax.ShapeDtypeStruct((N, 40), jnp.float32),
            jax.ShapeDtypeStruct((N, 40), jnp.float32),
        ],
        scratch_shapes=[pltpu.VMEM((BB, 128), jnp.float32)],
    )(A, x, x, scale, w2a, w2b, b2, csd_img, img_feats)


def _splat16(v):
    return jnp.full((16,), v, jnp.int32)


def kernel(adj_norm, adj_ori, feats_ori, img_feats, csd_ori, csd_img,
           Wb, Wm, Wl, W1, b1, W2, b2):
    B1 = _mm(img_feats, Wb)
    hidden = _bigmm(adj_norm, B1)
    B2 = _mm(hidden, Wm)
    mean = _bigmm(adj_norm, B2, relu=True)
    adj_logits, mx = _logits(mean)
    maxv = jnp.max(mx).reshape(1)

    keys, cnt = _keys(adj_logits, adj_ori, maxv)
    n_edges = cnt[0, 0]
    n_change = n_edges // 2

    keys3d = keys.reshape(NW, NCHUNK, CHUNK)

    par1 = jnp.concatenate([_splat16(0), _splat16(0),
                            _splat16(31), _splat16(20)])
    h1 = _sc_hist(keys3d, par1, masked=False)
    s1 = _search(h1, jnp.stack([n_change, n_change]), stage1=True)
    p1_rm, r2_rm, p1_add, r2_add = s1[0, 0], s1[1, 0], s1[2, 0], s1[3, 0]
    nnz_rm, nnz_add, n_rm, n_add = s1[4, 0], s1[5, 0], s1[6, 0], s1[7, 0]

    par2 = jnp.concatenate([_splat16(p1_rm), _splat16(2048 + p1_add),
                            _splat16(20), _splat16(10)])
    h2 = _sc_hist(keys3d, par2, masked=True)
    s2 = _search(h2, jnp.stack([r2_rm, r2_add]), stage1=False)
    p2_rm, r3_rm, p2_add, r3_add = s2[0, 0], s2[1, 0], s2[2, 0], s2[3, 0]

    pre_rm = (p1_rm << 10) | p2_rm
    pre_add = (p1_add << 10) | p2_add
    par3 = jnp.concatenate([_splat16(pre_rm), _splat16((1 << 21) | pre_add),
                            _splat16(10), _splat16(0)])
    h3 = _sc_hist(keys3d, par3, masked=True)
    s3 = _search(h3, jnp.stack([r3_rm, r3_add]), stage1=False)
    p3_rm, p3_add = s3[0, 0], s3[2, 0]

    bits_rm = (pre_rm << 10) | p3_rm
    bits_add = (pre_add << 10) | p3_add
    thresh_rm = jnp.where(nnz_rm > 0,
                          lax.bitcast_convert_type(bits_rm, jnp.float32), 0.0)
    thresh_add = jnp.where(nnz_add > 0,
                           lax.bitcast_convert_type(bits_add, jnp.float32), 0.0)

    scal = jnp.stack([thresh_rm, thresh_add,
                      (n_rm > 0).astype(jnp.float32),
                      (n_add > 0).astype(jnp.float32)])

    A, rs_abs, rs = _build(keys, adj_ori, scal)
    denom = jnp.maximum(rs_abs, 1e-12)
    deg = rs / denom + 1e-7
    scale = 1.0 / (denom * deg)

    w1a, w1b = W1[:256], W1[256:]
    w2a, w2b = W2[:128], W2[128:]
    x = _gcn1(A, feats_ori, scale, w1a, w1b, b1.reshape(1, 128))
    feat_total, preds_total, preds_img = _gcn2(
        A, x, scale, w2a, w2b, b2.reshape(1, 128), csd_img, img_feats)

    return preds_total, feat_total, preds_img, adj_logits
